# Initial kernel scaffold; baseline (speedup 1.0000x reference)
import jax
import jax.numpy as jnp
from jax.experimental import pallas as pl


def kernel(predict, target):
    raise NotImplementedError("write your pallas kernel here")


# speedup vs baseline: 33.3554x; 33.3554x over previous
import functools

import jax
import jax.numpy as jnp
from jax import lax
from jax.experimental import pallas as pl
from jax.experimental.pallas import tpu as pltpu

_THRESH = 0.7
_MIN_KEPT = 100000


def _fused_body(pred_ref, tgt_ref, stats_ref, *rest, emit_pn):
    x = pred_ref[0]
    t = tgt_ref[0]
    thr = jnp.float32(_THRESH)
    m = jnp.max(x, axis=0)
    e = jnp.exp(x - m[None])
    s = jnp.sum(e, axis=0)
    cls = lax.broadcasted_iota(jnp.int32, x.shape, 0)
    onehot = cls == t[None]
    tl = jnp.sum(jnp.where(onehot, x, 0.0), axis=0)
    el = jnp.sum(jnp.where(onehot, e, 0.0), axis=0)
    p = el / s
    nll = m + jnp.log(s) - tl
    lt = p < thr
    tot = jnp.sum(jnp.where(lt, nll, 0.0))
    c_lt = jnp.sum(lt.astype(jnp.float32))
    c_le = jnp.sum((p <= thr).astype(jnp.float32))

    first = (pl.program_id(0) == 0) & (pl.program_id(1) == 0)

    @pl.when(first)
    def _():
        stats_ref[0, 0] = tot
        stats_ref[0, 1] = c_lt
        stats_ref[0, 2] = c_le

    @pl.when(jnp.logical_not(first))
    def _():
        stats_ref[0, 0] += tot
        stats_ref[0, 1] += c_lt
        stats_ref[0, 2] += c_le

    if emit_pn:
        p_ref, nll_ref = rest
        p_ref[0] = p
        nll_ref[0] = nll


def _run_fused(predict, target, emit_pn, bh):
    n, c, h, w = predict.shape
    grid = (n, h // bh)
    in_specs = [
        pl.BlockSpec((1, c, bh, w), lambda i, j: (i, 0, j, 0)),
        pl.BlockSpec((1, bh, w), lambda i, j: (i, j, 0)),
    ]
    out_shapes = [jax.ShapeDtypeStruct((1, 3), jnp.float32)]
    out_specs = [pl.BlockSpec((1, 3), lambda i, j: (0, 0), memory_space=pltpu.SMEM)]
    if emit_pn:
        out_shapes += [
            jax.ShapeDtypeStruct((n, h, w), jnp.float32),
            jax.ShapeDtypeStruct((n, h, w), jnp.float32),
        ]
        out_specs += [
            pl.BlockSpec((1, bh, w), lambda i, j: (i, j, 0)),
            pl.BlockSpec((1, bh, w), lambda i, j: (i, j, 0)),
        ]
    return pl.pallas_call(
        functools.partial(_fused_body, emit_pn=emit_pn),
        grid=grid,
        in_specs=in_specs,
        out_specs=out_specs,
        out_shape=out_shapes,
    )(predict, target)


def _reduce_body(thr_ref, p_ref, nll_ref, out_ref):
    thr = thr_ref[0, 0]
    p = p_ref[0]
    nll = nll_ref[0]
    lt = p < thr
    tot = jnp.sum(jnp.where(lt, nll, 0.0))
    cnt = jnp.sum(lt.astype(jnp.float32))
    first = (pl.program_id(0) == 0) & (pl.program_id(1) == 0)

    @pl.when(first)
    def _():
        out_ref[0, 0] = tot
        out_ref[0, 1] = cnt

    @pl.when(jnp.logical_not(first))
    def _():
        out_ref[0, 0] += tot
        out_ref[0, 1] += cnt


def _masked_reduce(p, nll, thr, bh):
    n, h, w = p.shape
    grid = (n, h // bh)
    out = pl.pallas_call(
        _reduce_body,
        grid=grid,
        in_specs=[
            pl.BlockSpec(memory_space=pltpu.SMEM),
            pl.BlockSpec((1, bh, w), lambda i, j: (i, j, 0)),
            pl.BlockSpec((1, bh, w), lambda i, j: (i, j, 0)),
        ],
        out_specs=pl.BlockSpec((1, 2), lambda i, j: (0, 0), memory_space=pltpu.SMEM),
        out_shape=jax.ShapeDtypeStruct((1, 2), jnp.float32),
    )(thr.reshape(1, 1), p, nll)
    return out[0, 0], out[0, 1]


def _final(total, count):
    return jnp.where(count > 0, total / jnp.maximum(count, 1.0), total)


def kernel(predict, target):
    n, c, h, w = predict.shape
    numel = n * h * w
    kept_idx = max(min(_MIN_KEPT * n, numel - 1), 0)
    bh = 64 if h % 64 == 0 else 16

    (stats,) = _run_fused(predict, target, emit_pn=False, bh=bh)
    tot7, clt7, cle7 = stats[0, 0], stats[0, 1], stats[0, 2]

    def common(_):
        return _final(tot7, clt7)

    def rare(_):
        _, p, nll = _run_fused(predict, target, emit_pn=True, bh=bh)

        def body(i, prefix):
            bit = 29 - i
            cand_bits = prefix | (jnp.int32(1) << bit)
            cand = lax.bitcast_convert_type(cand_bits, jnp.float32)
            _, cnt = _masked_reduce(p, nll, cand, bh)
            return jnp.where(cnt <= jnp.float32(kept_idx), cand_bits, prefix)

        v_bits = lax.fori_loop(0, 30, body, jnp.int32(0))
        v = lax.bitcast_convert_type(v_bits, jnp.float32)
        total, count = _masked_reduce(p, nll, v, bh)
        return _final(total, count)

    return lax.cond(cle7 >= jnp.float32(kept_idx + 1), common, rare, operand=None)
